# baseline (speedup 1.0000x reference)
import functools

import jax
import jax.numpy as jnp
from jax import lax
from jax.experimental import pallas as pl
from jax.experimental.pallas import tpu as pltpu
from jax.experimental.pallas import tpu_sc as plsc

MAX_ROWS = 8192
EMB = 1024
NC = 2
NS = 16
NW = NC * NS
B_PER_W = MAX_ROWS // NW
CHUNK = 64
N_CHUNKS = B_PER_W // CHUNK

_MESH = plsc.VectorSubcoreMesh(core_axis_name="c", subcore_axis_name="s")
_OUT = jax.ShapeDtypeStruct((MAX_ROWS, EMB), jnp.float32)


def _worker_id():
    return lax.axis_index("s") * NC + lax.axis_index("c")


def _copy_body(table_hbm, out_hbm, sem):
    base = _worker_id() * B_PER_W
    pltpu.async_copy(
        table_hbm.at[pl.ds(base, B_PER_W)],
        out_hbm.at[pl.ds(base, B_PER_W)],
        sem,
    ).wait()


_sc_copy = functools.partial(
    pl.kernel,
    out_type=_OUT,
    mesh=_MESH,
    scratch_types=[pltpu.SemaphoreType.DMA],
)(_copy_body)


def _gather_body(table_hbm, idx_hbm, out_hbm, idx_v, buf_v, sem):
    base = _worker_id() * B_PER_W
    pltpu.sync_copy(idx_hbm.at[pl.ds(base, B_PER_W)], idx_v)

    def chunk(g, _):
        pltpu.async_copy(
            table_hbm.at[idx_v.at[pl.ds(g * CHUNK, CHUNK)]], buf_v, sem
        ).wait()
        pltpu.sync_copy(buf_v, out_hbm.at[pl.ds(base + g * CHUNK, CHUNK)])
        return ()

    lax.fori_loop(0, N_CHUNKS, chunk, (), unroll=False)


_sc_gather = functools.partial(
    pl.kernel,
    out_type=_OUT,
    mesh=_MESH,
    scratch_types=[
        pltpu.VMEM((B_PER_W,), jnp.int32),
        pltpu.VMEM((CHUNK, EMB), jnp.float32),
        pltpu.SemaphoreType.DMA,
    ],
)(_gather_body)


def kernel(seq_len, table):
    n = table.shape[0]
    offset = jnp.asarray(seq_len, dtype=jnp.int32) - jnp.int32(n)
    idx = jnp.clip(jnp.arange(n, dtype=jnp.int32) + offset, 0, n - 1)
    return lax.cond(
        offset == 0,
        lambda t, i: _sc_copy(t),
        lambda t, i: _sc_gather(t, i),
        table, idx,
    )


# speedup vs baseline: 22.8584x; 22.8584x over previous
import functools

import jax
import jax.numpy as jnp
from jax import lax
from jax.experimental import pallas as pl
from jax.experimental.pallas import tpu as pltpu
from jax.experimental.pallas import tpu_sc as plsc

MAX_ROWS = 8192
EMB = 1024
NC = 2
NS = 16
NW = NC * NS
B_PER_W = MAX_ROWS // NW
CHUNK = 64
N_CHUNKS = B_PER_W // CHUNK

_MESH = plsc.VectorSubcoreMesh(core_axis_name="c", subcore_axis_name="s")
_OUT = jax.ShapeDtypeStruct((MAX_ROWS, EMB), jnp.float32)


def _worker_id():
    return lax.axis_index("s") * NC + lax.axis_index("c")


CCH = 32
N_CCH = B_PER_W // CCH
CBUF = 2


def _copy_body(table_hbm, out_hbm, buf0, buf1, gsem0, gsem1, wsem0, wsem1):
    bufs = (buf0, buf1)
    gsems = (gsem0, gsem1)
    wsems = (wsem0, wsem1)
    base = _worker_id() * B_PER_W

    def read(g):
        b = g % CBUF
        return pltpu.async_copy(
            table_hbm.at[pl.ds(base + g * CCH, CCH)], bufs[b], gsems[b]
        )

    def write(g):
        b = g % CBUF
        return pltpu.async_copy(
            bufs[b], out_hbm.at[pl.ds(base + g * CCH, CCH)], wsems[b]
        )

    reads = [None] * N_CCH
    writes = [None] * N_CCH
    reads[0] = read(0)
    for g in range(N_CCH):
        reads[g].wait()
        if g + 1 < N_CCH:
            if g - 1 >= 0:
                writes[g - 1].wait()
            reads[g + 1] = read(g + 1)
        writes[g] = write(g)
    writes[N_CCH - 2].wait()
    writes[N_CCH - 1].wait()


_sc_copy = functools.partial(
    pl.kernel,
    out_type=_OUT,
    mesh=_MESH,
    scratch_types=[
        pltpu.VMEM((CCH, EMB), jnp.float32),
        pltpu.VMEM((CCH, EMB), jnp.float32),
        pltpu.SemaphoreType.DMA,
        pltpu.SemaphoreType.DMA,
        pltpu.SemaphoreType.DMA,
        pltpu.SemaphoreType.DMA,
    ],
)(_copy_body)


def _gather_body(table_hbm, idx_hbm, out_hbm, idx_v, buf_v, sem):
    base = _worker_id() * B_PER_W
    pltpu.sync_copy(idx_hbm.at[pl.ds(base, B_PER_W)], idx_v)

    def chunk(g, _):
        pltpu.async_copy(
            table_hbm.at[idx_v.at[pl.ds(g * CHUNK, CHUNK)]], buf_v, sem
        ).wait()
        pltpu.sync_copy(buf_v, out_hbm.at[pl.ds(base + g * CHUNK, CHUNK)])
        return ()

    lax.fori_loop(0, N_CHUNKS, chunk, (), unroll=False)


_sc_gather = functools.partial(
    pl.kernel,
    out_type=_OUT,
    mesh=_MESH,
    scratch_types=[
        pltpu.VMEM((B_PER_W,), jnp.int32),
        pltpu.VMEM((CHUNK, EMB), jnp.float32),
        pltpu.SemaphoreType.DMA,
    ],
)(_gather_body)


def kernel(seq_len, table):
    n = table.shape[0]
    offset = jnp.asarray(seq_len, dtype=jnp.int32) - jnp.int32(n)
    idx = jnp.clip(jnp.arange(n, dtype=jnp.int32) + offset, 0, n - 1)
    return lax.cond(
        offset == 0,
        lambda t, i: _sc_copy(t),
        lambda t, i: _sc_gather(t, i),
        table, idx,
    )


# speedup vs baseline: 24.4669x; 1.0704x over previous
import functools

import jax
import jax.numpy as jnp
from jax import lax
from jax.experimental import pallas as pl
from jax.experimental.pallas import tpu as pltpu
from jax.experimental.pallas import tpu_sc as plsc

MAX_ROWS = 8192
EMB = 1024
NC = 2
NS = 16
NW = NC * NS
B_PER_W = MAX_ROWS // NW
CHUNK = 64
N_CHUNKS = B_PER_W // CHUNK

_MESH = plsc.VectorSubcoreMesh(core_axis_name="c", subcore_axis_name="s")
_OUT = jax.ShapeDtypeStruct((MAX_ROWS, EMB), jnp.float32)


def _worker_id():
    return lax.axis_index("s") * NC + lax.axis_index("c")


CCH = 64
N_CCH = B_PER_W // CCH


def _copy_body(table_hbm, out_hbm, shared, sem):
    sid = lax.axis_index("s")
    base = _worker_id() * B_PER_W

    def chunk(g, _):
        pltpu.async_copy(
            table_hbm.at[pl.ds(base + g * CCH, CCH)], shared.at[sid], sem
        ).wait()
        pltpu.async_copy(
            shared.at[sid], out_hbm.at[pl.ds(base + g * CCH, CCH)], sem
        ).wait()
        return ()

    lax.fori_loop(0, N_CCH, chunk, (), unroll=False)


_sc_copy = functools.partial(
    pl.kernel,
    out_type=_OUT,
    mesh=_MESH,
    scratch_types=[
        pltpu.VMEM_SHARED((NS, CCH, EMB), jnp.float32),
        pltpu.SemaphoreType.DMA,
    ],
)(_copy_body)


def _gather_body(table_hbm, idx_hbm, out_hbm, idx_v, buf_v, sem):
    base = _worker_id() * B_PER_W
    pltpu.sync_copy(idx_hbm.at[pl.ds(base, B_PER_W)], idx_v)

    def chunk(g, _):
        pltpu.async_copy(
            table_hbm.at[idx_v.at[pl.ds(g * CHUNK, CHUNK)]], buf_v, sem
        ).wait()
        pltpu.sync_copy(buf_v, out_hbm.at[pl.ds(base + g * CHUNK, CHUNK)])
        return ()

    lax.fori_loop(0, N_CHUNKS, chunk, (), unroll=False)


_sc_gather = functools.partial(
    pl.kernel,
    out_type=_OUT,
    mesh=_MESH,
    scratch_types=[
        pltpu.VMEM((B_PER_W,), jnp.int32),
        pltpu.VMEM((CHUNK, EMB), jnp.float32),
        pltpu.SemaphoreType.DMA,
    ],
)(_gather_body)


def kernel(seq_len, table):
    n = table.shape[0]
    offset = jnp.asarray(seq_len, dtype=jnp.int32) - jnp.int32(n)
    idx = jnp.clip(jnp.arange(n, dtype=jnp.int32) + offset, 0, n - 1)
    return lax.cond(
        offset == 0,
        lambda t, i: _sc_copy(t),
        lambda t, i: _sc_gather(t, i),
        table, idx,
    )
